# initial kernel scaffold (unmeasured)
import math

import jax
import jax.numpy as jnp
from jax import lax
from jax.experimental import pallas as pl
from jax.experimental.pallas import tpu as pltpu

N_DEV = 16
B_LOC = 2
SQ = 128
D = 512
H_LOC = 4
DH = 64
B_GLB = N_DEV * B_LOC
ROWS = B_GLB * SQ
ROWS_LOC = B_LOC * SQ
DH_LOC = H_LOC * DH


def kernel(x, Wq, Wk, Wv, Wo):
    def body(x_ref, wq_ref, wk_ref, wv_ref, wo_ref, out_ref,
             xg, qh, kh, vh, ctxh, part, rsb,
             ag_send, ag_recv, rs_send, rs_recv):
        my = lax.axis_index("i")
        left = lax.rem(my + N_DEV - 1, N_DEV)
        right = lax.rem(my + 1, N_DEV)

        barrier = pltpu.get_barrier_semaphore()
        for nbr in (left, right):
            pl.semaphore_signal(barrier, inc=1, device_id=(nbr,),
                                device_id_type=pl.DeviceIdType.MESH)
        pl.semaphore_wait(barrier, 2)

        for b in range(B_LOC):
            xg[pl.ds(my * ROWS_LOC + b * SQ, SQ), :] = x_ref[b, :, :]

        for h in range(N_DEV - 1):
            o_send = lax.rem(my - h + N_DEV, N_DEV)
            rdma = pltpu.make_async_remote_copy(
                src_ref=xg.at[pl.ds(o_send * ROWS_LOC, ROWS_LOC)],
                dst_ref=xg.at[pl.ds(o_send * ROWS_LOC, ROWS_LOC)],
                send_sem=ag_send.at[h],
                recv_sem=ag_recv.at[h],
                device_id=(right,),
                device_id_type=pl.DeviceIdType.MESH,
            )
            rdma.start()
            rdma.wait()

        xfull = xg[...]
        q = jnp.dot(xfull, wq_ref[...], preferred_element_type=jnp.float32)
        k = jnp.dot(xfull, wk_ref[...], preferred_element_type=jnp.float32)
        v = jnp.dot(xfull, wv_ref[...], preferred_element_type=jnp.float32)

        rows_i = lax.broadcasted_iota(jnp.int32, (ROWS, DH_LOC), 0)
        cols_i = lax.broadcasted_iota(jnp.int32, (ROWS, DH_LOC), 1)
        s_pos = jnp.remainder(rows_i, SQ).astype(jnp.float32)
        k2 = ((jnp.remainder(cols_i, DH) // 2) * 2).astype(jnp.float32)
        inv = jnp.exp(k2 * (-math.log(10000.0) / DH))
        ang = s_pos * inv
        cosv = jnp.cos(ang)
        sinv = jnp.sin(ang)

        ri = lax.broadcasted_iota(jnp.int32, (DH_LOC, DH_LOC), 0)
        ci = lax.broadcasted_iota(jnp.int32, (DH_LOC, DH_LOC), 1)
        even_c = jnp.remainder(ci, 2) == 0
        rot_m = (jnp.where(even_c & (ri == ci + 1), -1.0, 0.0)
                 + jnp.where((~even_c) & (ri == ci - 1), 1.0, 0.0)
                 ).astype(jnp.float32)

        q = q * cosv + jnp.dot(q, rot_m, preferred_element_type=jnp.float32) * sinv
        k = k * cosv + jnp.dot(k, rot_m, preferred_element_type=jnp.float32) * sinv

        for hh in range(H_LOC):
            qh[hh] = q[:, hh * DH:(hh + 1) * DH]
            kh[hh] = k[:, hh * DH:(hh + 1) * DH]
            vh[hh] = v[:, hh * DH:(hh + 1) * DH]

        blk_r = lax.broadcasted_iota(jnp.int32, (H_LOC * SQ, H_LOC * SQ), 0) // SQ
        blk_c = lax.broadcasted_iota(jnp.int32, (H_LOC * SQ, H_LOC * SQ), 1) // SQ
        same_head = blk_r == blk_c
        neg = jnp.float32(-1e30)

        def attn_body(b, carry):
            qs = qh[:, pl.ds(b * SQ, SQ), :].reshape(H_LOC * SQ, DH)
            ks = kh[:, pl.ds(b * SQ, SQ), :].reshape(H_LOC * SQ, DH)
            vs = vh[:, pl.ds(b * SQ, SQ), :].reshape(H_LOC * SQ, DH)
            s = lax.dot_general(qs, ks, (((1,), (1,)), ((), ())),
                                preferred_element_type=jnp.float32) * 0.125
            s = jnp.where(same_head, s, neg)
            m = jnp.max(s, axis=-1, keepdims=True)
            w = jnp.exp(s - m)
            w = w / jnp.sum(w, axis=-1, keepdims=True)
            ctx = jnp.dot(w, vs, preferred_element_type=jnp.float32)
            ctxh[:, pl.ds(b * SQ, SQ), :] = ctx.reshape(H_LOC, SQ, DH)
            return carry

        lax.fori_loop(0, B_GLB, attn_body, 0)

        acc = jnp.dot(ctxh[0], wo_ref[0:DH, :],
                      preferred_element_type=jnp.float32)
        for hh in range(1, H_LOC):
            acc = acc + jnp.dot(ctxh[hh], wo_ref[hh * DH:(hh + 1) * DH, :],
                                preferred_element_type=jnp.float32)
        part[...] = acc

        for t in range(N_DEV - 1):
            c_send = lax.rem(my - 1 - t + 2 * N_DEV, N_DEV)
            if t == 0:
                src = part.at[pl.ds(c_send * ROWS_LOC, ROWS_LOC)]
            else:
                src = rsb.at[t - 1]
            rdma = pltpu.make_async_remote_copy(
                src_ref=src,
                dst_ref=rsb.at[t],
                send_sem=rs_send.at[t],
                recv_sem=rs_recv.at[t],
                device_id=(right,),
                device_id_type=pl.DeviceIdType.MESH,
            )
            rdma.start()
            rdma.wait()
            c_recv = lax.rem(my - 2 - t + 2 * N_DEV, N_DEV)
            rsb[t] = rsb[t] + part[pl.ds(c_recv * ROWS_LOC, ROWS_LOC), :]

        for b in range(B_LOC):
            out_ref[b, :, :] = rsb[N_DEV - 2, b * SQ:(b + 1) * SQ, :]

    return pl.pallas_call(
        body,
        out_shape=jax.ShapeDtypeStruct((B_LOC, SQ, D), jnp.float32),
        in_specs=[pl.BlockSpec(memory_space=pltpu.VMEM)] * 5,
        out_specs=pl.BlockSpec(memory_space=pltpu.VMEM),
        scratch_shapes=[
            pltpu.VMEM((ROWS, D), jnp.float32),
            pltpu.VMEM((H_LOC, ROWS, DH), jnp.float32),
            pltpu.VMEM((H_LOC, ROWS, DH), jnp.float32),
            pltpu.VMEM((H_LOC, ROWS, DH), jnp.float32),
            pltpu.VMEM((H_LOC, ROWS, DH), jnp.float32),
            pltpu.VMEM((ROWS, D), jnp.float32),
            pltpu.VMEM((N_DEV - 1, ROWS_LOC, D), jnp.float32),
            pltpu.SemaphoreType.DMA((N_DEV - 1,)),
            pltpu.SemaphoreType.DMA((N_DEV - 1,)),
            pltpu.SemaphoreType.DMA((N_DEV - 1,)),
            pltpu.SemaphoreType.DMA((N_DEV - 1,)),
        ],
        compiler_params=pltpu.CompilerParams(collective_id=0),
    )(x, Wq, Wk, Wv, Wo)


# baseline (device time: 287828 ns/iter reference)
import math

import jax
import jax.numpy as jnp
from jax import lax
from jax.experimental import pallas as pl
from jax.experimental.pallas import tpu as pltpu

N_DEV = 16
B_LOC = 2
SQ = 128
D = 512
H_LOC = 4
DH = 64
B_GLB = N_DEV * B_LOC
ROWS = B_GLB * SQ
ROWS_LOC = B_LOC * SQ
DH_LOC = H_LOC * DH


def kernel(x, Wq, Wk, Wv, Wo):
    def body(x_ref, wq_ref, wk_ref, wv_ref, wo_ref, out_ref,
             xg, qb_ref, kb_ref, vb_ref, ctx_ref, rsb,
             ag_send, ag_recv, rs_send, rs_recv):
        my = lax.axis_index("i")
        left = lax.rem(my + N_DEV - 1, N_DEV)
        right = lax.rem(my + 1, N_DEV)

        barrier = pltpu.get_barrier_semaphore()
        for nbr in (left, right):
            pl.semaphore_signal(barrier, inc=1, device_id=(nbr,),
                                device_id_type=pl.DeviceIdType.MESH)
        pl.semaphore_wait(barrier, 2)

        for b in range(B_LOC):
            xg[pl.ds(my * ROWS_LOC + b * SQ, SQ), :] = x_ref[b, :, :]

        for h in range(N_DEV - 1):
            o_send = lax.rem(my - h + N_DEV, N_DEV)
            rdma = pltpu.make_async_remote_copy(
                src_ref=xg.at[pl.ds(o_send * ROWS_LOC, ROWS_LOC)],
                dst_ref=xg.at[pl.ds(o_send * ROWS_LOC, ROWS_LOC)],
                send_sem=ag_send.at[h],
                recv_sem=ag_recv.at[h],
                device_id=(right,),
                device_id_type=pl.DeviceIdType.MESH,
            )
            rdma.start()
            rdma.wait()

        qb_ref[...] = jnp.dot(xg[...], wq_ref[...],
                              preferred_element_type=jnp.float32)
        kb_ref[...] = jnp.dot(xg[...], wk_ref[...],
                              preferred_element_type=jnp.float32)
        vb_ref[...] = jnp.dot(xg[...], wv_ref[...],
                              preferred_element_type=jnp.float32)

        rows_i = lax.broadcasted_iota(jnp.int32, (SQ, DH_LOC), 0)
        cols_i = lax.broadcasted_iota(jnp.int32, (SQ, DH_LOC), 1)
        k2 = ((jnp.remainder(cols_i, DH) // 2) * 2).astype(jnp.float32)
        inv = jnp.exp(k2 * (-math.log(10000.0) / DH))
        ang = rows_i.astype(jnp.float32) * inv
        cosv = jnp.cos(ang)
        sinv = jnp.sin(ang)

        ri = lax.broadcasted_iota(jnp.int32, (DH_LOC, DH_LOC), 0)
        ci = lax.broadcasted_iota(jnp.int32, (DH_LOC, DH_LOC), 1)
        even_c = jnp.remainder(ci, 2) == 0
        rot_m = (jnp.where(even_c & (ri == ci + 1), -1.0, 0.0)
                 + jnp.where((~even_c) & (ri == ci - 1), 1.0, 0.0)
                 ).astype(jnp.float32)

        neg = jnp.float32(-1e30)

        def attn_body(b, carry):
            qb = qb_ref[pl.ds(b * SQ, SQ), :]
            kb = kb_ref[pl.ds(b * SQ, SQ), :]
            vb = vb_ref[pl.ds(b * SQ, SQ), :]
            qb = qb * cosv + jnp.dot(qb, rot_m,
                                     preferred_element_type=jnp.float32) * sinv
            kb = kb * cosv + jnp.dot(kb, rot_m,
                                     preferred_element_type=jnp.float32) * sinv
            ctx_heads = []
            for hh in range(H_LOC):
                qs = qb[:, hh * DH:(hh + 1) * DH]
                ks = kb[:, hh * DH:(hh + 1) * DH]
                vs = vb[:, hh * DH:(hh + 1) * DH]
                s = lax.dot_general(qs, ks, (((1,), (1,)), ((), ())),
                                    preferred_element_type=jnp.float32) * 0.125
                m = jnp.max(s, axis=-1, keepdims=True)
                w = jnp.exp(s - m)
                w = w / jnp.sum(w, axis=-1, keepdims=True)
                ctx_heads.append(jnp.dot(w, vs,
                                         preferred_element_type=jnp.float32))
            ctx_ref[pl.ds(b * SQ, SQ), :] = jnp.concatenate(ctx_heads, axis=1)
            return carry

        lax.fori_loop(0, B_GLB, attn_body, 0)

        part = xg
        acc = jnp.dot(ctx_ref[:, 0:DH], wo_ref[0:DH, :],
                      preferred_element_type=jnp.float32)
        for hh in range(1, H_LOC):
            acc = acc + jnp.dot(ctx_ref[:, hh * DH:(hh + 1) * DH],
                                wo_ref[hh * DH:(hh + 1) * DH, :],
                                preferred_element_type=jnp.float32)
        part[...] = acc

        for t in range(N_DEV - 1):
            c_send = lax.rem(my - 1 - t + 2 * N_DEV, N_DEV)
            if t == 0:
                src = part.at[pl.ds(c_send * ROWS_LOC, ROWS_LOC)]
            else:
                src = rsb.at[t - 1]
            rdma = pltpu.make_async_remote_copy(
                src_ref=src,
                dst_ref=rsb.at[t],
                send_sem=rs_send.at[t],
                recv_sem=rs_recv.at[t],
                device_id=(right,),
                device_id_type=pl.DeviceIdType.MESH,
            )
            rdma.start()
            rdma.wait()
            c_recv = lax.rem(my - 2 - t + 2 * N_DEV, N_DEV)
            rsb[t] = rsb[t] + part[pl.ds(c_recv * ROWS_LOC, ROWS_LOC), :]

        for b in range(B_LOC):
            out_ref[b, :, :] = rsb[N_DEV - 2, b * SQ:(b + 1) * SQ, :]

    return pl.pallas_call(
        body,
        out_shape=jax.ShapeDtypeStruct((B_LOC, SQ, D), jnp.float32),
        in_specs=[pl.BlockSpec(memory_space=pltpu.VMEM)] * 5,
        out_specs=pl.BlockSpec(memory_space=pltpu.VMEM),
        scratch_shapes=[
            pltpu.VMEM((ROWS, D), jnp.float32),
            pltpu.VMEM((ROWS, DH_LOC), jnp.float32),
            pltpu.VMEM((ROWS, DH_LOC), jnp.float32),
            pltpu.VMEM((ROWS, DH_LOC), jnp.float32),
            pltpu.VMEM((ROWS, DH_LOC), jnp.float32),
            pltpu.VMEM((N_DEV - 1, ROWS_LOC, D), jnp.float32),
            pltpu.SemaphoreType.DMA((N_DEV - 1,)),
            pltpu.SemaphoreType.DMA((N_DEV - 1,)),
            pltpu.SemaphoreType.DMA((N_DEV - 1,)),
            pltpu.SemaphoreType.DMA((N_DEV - 1,)),
        ],
        compiler_params=pltpu.CompilerParams(
            collective_id=0,
            vmem_limit_bytes=63 * 1024 * 1024,
        ),
    )(x, Wq, Wk, Wv, Wo)


# device time: 154269 ns/iter; 1.8658x vs baseline; 1.8658x over previous
import math

import jax
import jax.numpy as jnp
from jax import lax
from jax.experimental import pallas as pl
from jax.experimental.pallas import tpu as pltpu

N_DEV = 16
B_LOC = 2
SQ = 128
D = 512
H_LOC = 4
DH = 64
B_GLB = N_DEV * B_LOC
ROWS = B_GLB * SQ
DH_LOC = H_LOC * DH


def kernel(x, Wq, Wk, Wv, Wo):
    def body(x_ref, wq_ref, wk_ref, wv_ref, wo_ref, out_ref,
             xg, part, rsb_r, rsb_l,
             agr_send, agr_recv, agl_send, agl_recv,
             rsr_send, rsr_recv, rsl_send, rsl_recv):
        my = lax.axis_index("i")
        left = lax.rem(my + N_DEV - 1, N_DEV)
        right = lax.rem(my + 1, N_DEV)

        def m16(v):
            return lax.rem(v + 2 * N_DEV, N_DEV)

        def erow(c):
            return c * (2 * SQ)

        def orow(c):
            return c * (2 * SQ) + SQ

        barrier = pltpu.get_barrier_semaphore()
        for nbr in (left, right):
            pl.semaphore_signal(barrier, inc=1, device_id=(nbr,),
                                device_id_type=pl.DeviceIdType.MESH)
        pl.semaphore_wait(barrier, 2)

        xg[pl.ds(erow(my), SQ), :] = x_ref[0, :, :]
        xg[pl.ds(orow(my), SQ), :] = x_ref[1, :, :]

        rows_i = lax.broadcasted_iota(jnp.int32, (SQ, DH_LOC), 0)
        cols_i = lax.broadcasted_iota(jnp.int32, (SQ, DH_LOC), 1)
        k2 = ((jnp.remainder(cols_i, DH) // 2) * 2).astype(jnp.float32)
        inv = jnp.exp(k2 * (-math.log(10000.0) / DH))
        ang = rows_i.astype(jnp.float32) * inv
        cosv = jnp.cos(ang)
        sinv = jnp.sin(ang)

        ri = lax.broadcasted_iota(jnp.int32, (DH_LOC, DH_LOC), 0)
        ci = lax.broadcasted_iota(jnp.int32, (DH_LOC, DH_LOC), 1)
        even_c = jnp.remainder(ci, 2) == 0
        rot_m = (jnp.where(even_c & (ri == ci + 1), -1.0, 0.0)
                 + jnp.where((~even_c) & (ri == ci - 1), 1.0, 0.0)
                 ).astype(jnp.float32)

        wq = wq_ref[...]
        wk = wk_ref[...]
        wv = wv_ref[...]
        wo = wo_ref[...]

        def compute_partial(row):
            xb = xg[pl.ds(row, SQ), :]
            q = jnp.dot(xb, wq, preferred_element_type=jnp.float32)
            k = jnp.dot(xb, wk, preferred_element_type=jnp.float32)
            v = jnp.dot(xb, wv, preferred_element_type=jnp.float32)
            q = q * cosv + jnp.dot(q, rot_m,
                                   preferred_element_type=jnp.float32) * sinv
            k = k * cosv + jnp.dot(k, rot_m,
                                   preferred_element_type=jnp.float32) * sinv
            ctx_heads = []
            for hh in range(H_LOC):
                qs = q[:, hh * DH:(hh + 1) * DH]
                ks = k[:, hh * DH:(hh + 1) * DH]
                vs = v[:, hh * DH:(hh + 1) * DH]
                s = lax.dot_general(qs, ks, (((1,), (1,)), ((), ())),
                                    preferred_element_type=jnp.float32) * 0.125
                mx = jnp.max(s, axis=-1, keepdims=True)
                w = jnp.exp(s - mx)
                w = w / jnp.sum(w, axis=-1, keepdims=True)
                ctx_heads.append(jnp.dot(w, vs,
                                         preferred_element_type=jnp.float32))
            ctx = jnp.concatenate(ctx_heads, axis=1)
            part[pl.ds(row, SQ), :] = jnp.dot(
                ctx, wo, preferred_element_type=jnp.float32)

        def remote(src, dst, ssem, rsem, dev):
            return pltpu.make_async_remote_copy(
                src_ref=src, dst_ref=dst, send_sem=ssem, recv_sem=rsem,
                device_id=(dev,), device_id_type=pl.DeviceIdType.MESH)

        for h in range(N_DEV):
            if h < N_DEV - 1:
                ag_r = remote(xg.at[pl.ds(erow(m16(my - h)), SQ)],
                              xg.at[pl.ds(erow(m16(my - h)), SQ)],
                              agr_send.at[h], agr_recv.at[h], right)
                ag_r.start()
                ag_l = remote(xg.at[pl.ds(orow(m16(my + h)), SQ)],
                              xg.at[pl.ds(orow(m16(my + h)), SQ)],
                              agl_send.at[h], agl_recv.at[h], left)
                ag_l.start()
            if h >= 1:
                t = h - 1
                src_r = (part.at[pl.ds(erow(m16(my - 1)), SQ)] if t == 0
                         else rsb_r.at[t - 1])
                rs_r = remote(src_r, rsb_r.at[t],
                              rsr_send.at[t], rsr_recv.at[t], right)
                rs_r.start()
                src_l = (part.at[pl.ds(orow(m16(my + 1)), SQ)] if t == 0
                         else rsb_l.at[t - 1])
                rs_l = remote(src_l, rsb_l.at[t],
                              rsl_send.at[t], rsl_recv.at[t], left)
                rs_l.start()

            if h < N_DEV - 1:
                ag_r.wait_recv()
                compute_partial(erow(m16(my - h - 1)))
                ag_l.wait_recv()
                compute_partial(orow(m16(my + h + 1)))
            else:
                compute_partial(erow(my))
                compute_partial(orow(my))

            if h >= 1:
                t = h - 1
                rs_r.wait_recv()
                rsb_r[t] = rsb_r[t] + part[pl.ds(erow(m16(my - 2 - t)), SQ), :]
                rs_l.wait_recv()
                rsb_l[t] = rsb_l[t] + part[pl.ds(orow(m16(my + 2 + t)), SQ), :]

            if h < N_DEV - 1:
                ag_r.wait_send()
                ag_l.wait_send()
            if h >= 1:
                rs_r.wait_send()
                rs_l.wait_send()

        out_ref[0, :, :] = rsb_r[N_DEV - 2]
        out_ref[1, :, :] = rsb_l[N_DEV - 2]

    return pl.pallas_call(
        body,
        out_shape=jax.ShapeDtypeStruct((B_LOC, SQ, D), jnp.float32),
        in_specs=[pl.BlockSpec(memory_space=pltpu.VMEM)] * 5,
        out_specs=pl.BlockSpec(memory_space=pltpu.VMEM),
        scratch_shapes=[
            pltpu.VMEM((ROWS, D), jnp.float32),
            pltpu.VMEM((ROWS, D), jnp.float32),
            pltpu.VMEM((N_DEV - 1, SQ, D), jnp.float32),
            pltpu.VMEM((N_DEV - 1, SQ, D), jnp.float32),
            pltpu.SemaphoreType.DMA((N_DEV - 1,)),
            pltpu.SemaphoreType.DMA((N_DEV - 1,)),
            pltpu.SemaphoreType.DMA((N_DEV - 1,)),
            pltpu.SemaphoreType.DMA((N_DEV - 1,)),
            pltpu.SemaphoreType.DMA((N_DEV - 1,)),
            pltpu.SemaphoreType.DMA((N_DEV - 1,)),
            pltpu.SemaphoreType.DMA((N_DEV - 1,)),
            pltpu.SemaphoreType.DMA((N_DEV - 1,)),
        ],
        compiler_params=pltpu.CompilerParams(
            collective_id=0,
            vmem_limit_bytes=63 * 1024 * 1024,
        ),
    )(x, Wq, Wk, Wv, Wo)


# device time: 147680 ns/iter; 1.9490x vs baseline; 1.0446x over previous
import math

import jax
import jax.numpy as jnp
from jax import lax
from jax.experimental import pallas as pl
from jax.experimental.pallas import tpu as pltpu

N_DEV = 16
B_LOC = 2
SQ = 128
D = 512
H_LOC = 4
DH = 64
B_GLB = N_DEV * B_LOC
ROWS = B_GLB * SQ
DH_LOC = H_LOC * DH


def kernel(x, Wq, Wk, Wv, Wo):
    def body(x_ref, wq_ref, wk_ref, wv_ref, wo_ref, out_ref,
             xg, part, rsb_r, rsb_l,
             agr_send, agr_recv, agl_send, agl_recv,
             rsr_send, rsr_recv, rsl_send, rsl_recv):
        my = lax.axis_index("i")
        left = lax.rem(my + N_DEV - 1, N_DEV)
        right = lax.rem(my + 1, N_DEV)

        def m16(v):
            return lax.rem(v + 2 * N_DEV, N_DEV)

        def erow(c):
            return c * (2 * SQ)

        def orow(c):
            return c * (2 * SQ) + SQ

        barrier = pltpu.get_barrier_semaphore()
        for nbr in (left, right):
            pl.semaphore_signal(barrier, inc=1, device_id=(nbr,),
                                device_id_type=pl.DeviceIdType.MESH)
        pl.semaphore_wait(barrier, 2)

        xg[pl.ds(erow(my), SQ), :] = x_ref[0, :, :]
        xg[pl.ds(orow(my), SQ), :] = x_ref[1, :, :]

        rows_i = lax.broadcasted_iota(jnp.int32, (SQ, DH_LOC), 0)
        cols_i = lax.broadcasted_iota(jnp.int32, (SQ, DH_LOC), 1)
        k2 = ((jnp.remainder(cols_i, DH) // 2) * 2).astype(jnp.float32)
        inv = jnp.exp(k2 * (-math.log(10000.0) / DH))
        ang = rows_i.astype(jnp.float32) * inv
        cosv = jnp.cos(ang)
        sinv = jnp.sin(ang)

        ri = lax.broadcasted_iota(jnp.int32, (DH_LOC, DH_LOC), 0)
        ci = lax.broadcasted_iota(jnp.int32, (DH_LOC, DH_LOC), 1)
        even_c = jnp.remainder(ci, 2) == 0
        rot_m = (jnp.where(even_c & (ri == ci + 1), -1.0, 0.0)
                 + jnp.where((~even_c) & (ri == ci - 1), 1.0, 0.0)
                 ).astype(jnp.float32)

        cos2 = jnp.concatenate([cosv, cosv], axis=0)
        sin2 = jnp.concatenate([sinv, sinv], axis=0)

        wq = wq_ref[...]
        wk = wk_ref[...]
        wv = wv_ref[...]
        wo = wo_ref[...]

        def compute_partial(row_e, row_o):
            xb = jnp.concatenate(
                [xg[pl.ds(row_e, SQ), :], xg[pl.ds(row_o, SQ), :]], axis=0)
            q = jnp.dot(xb, wq, preferred_element_type=jnp.float32)
            k = jnp.dot(xb, wk, preferred_element_type=jnp.float32)
            v = jnp.dot(xb, wv, preferred_element_type=jnp.float32)
            q = q * cos2 + jnp.dot(q, rot_m,
                                   preferred_element_type=jnp.float32) * sin2
            k = k * cos2 + jnp.dot(k, rot_m,
                                   preferred_element_type=jnp.float32) * sin2
            halves = []
            for bb in range(2):
                ctx_heads = []
                for hh in range(H_LOC):
                    qs = q[bb * SQ:(bb + 1) * SQ, hh * DH:(hh + 1) * DH]
                    ks = k[bb * SQ:(bb + 1) * SQ, hh * DH:(hh + 1) * DH]
                    vs = v[bb * SQ:(bb + 1) * SQ, hh * DH:(hh + 1) * DH]
                    s = lax.dot_general(
                        qs, ks, (((1,), (1,)), ((), ())),
                        preferred_element_type=jnp.float32) * 0.125
                    mx = jnp.max(s, axis=-1, keepdims=True)
                    w = jnp.exp(s - mx)
                    w = w / jnp.sum(w, axis=-1, keepdims=True)
                    ctx_heads.append(jnp.dot(
                        w, vs, preferred_element_type=jnp.float32))
                halves.append(jnp.concatenate(ctx_heads, axis=1))
            ctx = jnp.concatenate(halves, axis=0)
            po = jnp.dot(ctx, wo, preferred_element_type=jnp.float32)
            part[pl.ds(row_e, SQ), :] = po[0:SQ, :]
            part[pl.ds(row_o, SQ), :] = po[SQ:2 * SQ, :]

        def remote(src, dst, ssem, rsem, dev):
            return pltpu.make_async_remote_copy(
                src_ref=src, dst_ref=dst, send_sem=ssem, recv_sem=rsem,
                device_id=(dev,), device_id_type=pl.DeviceIdType.MESH)

        for h in range(N_DEV + 1):
            if h < N_DEV - 1:
                ag_r = remote(xg.at[pl.ds(erow(m16(my - h)), SQ)],
                              xg.at[pl.ds(erow(m16(my - h)), SQ)],
                              agr_send.at[h], agr_recv.at[h], right)
                ag_r.start()
                ag_l = remote(xg.at[pl.ds(orow(m16(my + h)), SQ)],
                              xg.at[pl.ds(orow(m16(my + h)), SQ)],
                              agl_send.at[h], agl_recv.at[h], left)
                ag_l.start()
            if h >= 2:
                t = h - 2
                src_r = (part.at[pl.ds(erow(m16(my - 1)), SQ)] if t == 0
                         else rsb_r.at[t - 1])
                rs_r = remote(src_r, rsb_r.at[t],
                              rsr_send.at[t], rsr_recv.at[t], right)
                rs_r.start()
                src_l = (part.at[pl.ds(orow(m16(my + 1)), SQ)] if t == 0
                         else rsb_l.at[t - 1])
                rs_l = remote(src_l, rsb_l.at[t],
                              rsl_send.at[t], rsl_recv.at[t], left)
                rs_l.start()

            if h < N_DEV:
                compute_partial(erow(m16(my - h)), orow(m16(my + h)))

            if h >= 2:
                t = h - 2
                rs_r.wait_recv()
                rsb_r[t] = rsb_r[t] + part[pl.ds(erow(m16(my - 2 - t)), SQ), :]
                rs_l.wait_recv()
                rsb_l[t] = rsb_l[t] + part[pl.ds(orow(m16(my + 2 + t)), SQ), :]
                rs_r.wait_send()
                rs_l.wait_send()

            if h < N_DEV - 1:
                ag_r.wait_recv()
                ag_l.wait_recv()
                ag_r.wait_send()
                ag_l.wait_send()

        out_ref[0, :, :] = rsb_r[N_DEV - 2]
        out_ref[1, :, :] = rsb_l[N_DEV - 2]

    return pl.pallas_call(
        body,
        out_shape=jax.ShapeDtypeStruct((B_LOC, SQ, D), jnp.float32),
        in_specs=[pl.BlockSpec(memory_space=pltpu.VMEM)] * 5,
        out_specs=pl.BlockSpec(memory_space=pltpu.VMEM),
        scratch_shapes=[
            pltpu.VMEM((ROWS, D), jnp.float32),
            pltpu.VMEM((ROWS, D), jnp.float32),
            pltpu.VMEM((N_DEV - 1, SQ, D), jnp.float32),
            pltpu.VMEM((N_DEV - 1, SQ, D), jnp.float32),
            pltpu.SemaphoreType.DMA((N_DEV - 1,)),
            pltpu.SemaphoreType.DMA((N_DEV - 1,)),
            pltpu.SemaphoreType.DMA((N_DEV - 1,)),
            pltpu.SemaphoreType.DMA((N_DEV - 1,)),
            pltpu.SemaphoreType.DMA((N_DEV - 1,)),
            pltpu.SemaphoreType.DMA((N_DEV - 1,)),
            pltpu.SemaphoreType.DMA((N_DEV - 1,)),
            pltpu.SemaphoreType.DMA((N_DEV - 1,)),
        ],
        compiler_params=pltpu.CompilerParams(
            collective_id=0,
            vmem_limit_bytes=63 * 1024 * 1024,
        ),
    )(x, Wq, Wk, Wv, Wo)


# device time: 147010 ns/iter; 1.9579x vs baseline; 1.0046x over previous
import math

import jax
import jax.numpy as jnp
from jax import lax
from jax.experimental import pallas as pl
from jax.experimental.pallas import tpu as pltpu

N_DEV = 16
B_LOC = 2
SQ = 128
D = 512
H_LOC = 4
DH = 64
B_GLB = N_DEV * B_LOC
ROWS = B_GLB * SQ
DH_LOC = H_LOC * DH


def kernel(x, Wq, Wk, Wv, Wo):
    def body(x_ref, wq_ref, wk_ref, wv_ref, wo_ref, out_ref,
             xg, part, rsb_r, rsb_l,
             agr_send, agr_recv, agl_send, agl_recv,
             rsr_send, rsr_recv, rsl_send, rsl_recv):
        my = lax.axis_index("i")
        left = lax.rem(my + N_DEV - 1, N_DEV)
        right = lax.rem(my + 1, N_DEV)

        def m16(v):
            return lax.rem(v + 2 * N_DEV, N_DEV)

        def erow(c):
            return c * (2 * SQ)

        def orow(c):
            return c * (2 * SQ) + SQ

        barrier = pltpu.get_barrier_semaphore()
        for nbr in (left, right):
            pl.semaphore_signal(barrier, inc=1, device_id=(nbr,),
                                device_id_type=pl.DeviceIdType.MESH)
        pl.semaphore_wait(barrier, 2)

        xg[pl.ds(erow(my), SQ), :] = x_ref[0, :, :]
        xg[pl.ds(orow(my), SQ), :] = x_ref[1, :, :]

        rows_i = lax.broadcasted_iota(jnp.int32, (SQ, DH_LOC), 0)
        cols_i = lax.broadcasted_iota(jnp.int32, (SQ, DH_LOC), 1)
        k2 = ((jnp.remainder(cols_i, DH) // 2) * 2).astype(jnp.float32)
        inv = jnp.exp(k2 * (-math.log(10000.0) / DH))
        ang = rows_i.astype(jnp.float32) * inv
        cosv = jnp.cos(ang)
        sinv = jnp.sin(ang)

        ri = lax.broadcasted_iota(jnp.int32, (DH_LOC, DH_LOC), 0)
        ci = lax.broadcasted_iota(jnp.int32, (DH_LOC, DH_LOC), 1)
        even_c = jnp.remainder(ci, 2) == 0
        rot_m = (jnp.where(even_c & (ri == ci + 1), -1.0, 0.0)
                 + jnp.where((~even_c) & (ri == ci - 1), 1.0, 0.0)
                 ).astype(jnp.float32)

        cos2 = jnp.concatenate([cosv, cosv], axis=0)
        sin2 = jnp.concatenate([sinv, sinv], axis=0)

        wq = wq_ref[...]
        wk = wk_ref[...]
        wv = wv_ref[...]
        wo = wo_ref[...]

        def compute_partial(row_e, row_o):
            xb = jnp.concatenate(
                [xg[pl.ds(row_e, SQ), :], xg[pl.ds(row_o, SQ), :]], axis=0)
            q = jnp.dot(xb, wq, preferred_element_type=jnp.float32)
            k = jnp.dot(xb, wk, preferred_element_type=jnp.float32)
            v = jnp.dot(xb, wv, preferred_element_type=jnp.float32)
            q = q * cos2 + jnp.dot(q, rot_m,
                                   preferred_element_type=jnp.float32) * sin2
            k = k * cos2 + jnp.dot(k, rot_m,
                                   preferred_element_type=jnp.float32) * sin2
            halves = []
            for bb in range(2):
                ctx_heads = []
                for hh in range(H_LOC):
                    qs = q[bb * SQ:(bb + 1) * SQ, hh * DH:(hh + 1) * DH]
                    ks = k[bb * SQ:(bb + 1) * SQ, hh * DH:(hh + 1) * DH]
                    vs = v[bb * SQ:(bb + 1) * SQ, hh * DH:(hh + 1) * DH]
                    s = lax.dot_general(
                        qs, ks, (((1,), (1,)), ((), ())),
                        preferred_element_type=jnp.float32) * 0.125
                    mx = jnp.max(s, axis=-1, keepdims=True)
                    w = jnp.exp(s - mx)
                    w = w / jnp.sum(w, axis=-1, keepdims=True)
                    ctx_heads.append(jnp.dot(
                        w, vs, preferred_element_type=jnp.float32))
                halves.append(jnp.concatenate(ctx_heads, axis=1))
            ctx = jnp.concatenate(halves, axis=0)
            po = jnp.dot(ctx, wo, preferred_element_type=jnp.float32)
            part[pl.ds(row_e, SQ), :] = po[0:SQ, :]
            part[pl.ds(row_o, SQ), :] = po[SQ:2 * SQ, :]

        def remote(src, dst, ssem, rsem, dev):
            return pltpu.make_async_remote_copy(
                src_ref=src, dst_ref=dst, send_sem=ssem, recv_sem=rsem,
                device_id=(dev,), device_id_type=pl.DeviceIdType.MESH)

        HD = D // 2

        agr_d = agl_d = rsr_d = rsl_d = None
        for h in range(N_DEV + 1):
            if h < N_DEV - 1:
                new_agr, new_agl = [], []
                for cc in range(2):
                    if agr_d is not None:
                        agr_d[cc].wait_recv()
                    dr = remote(
                        xg.at[pl.ds(erow(m16(my - h)), SQ), pl.ds(cc * HD, HD)],
                        xg.at[pl.ds(erow(m16(my - h)), SQ), pl.ds(cc * HD, HD)],
                        agr_send.at[h, cc], agr_recv.at[h, cc], right)
                    dr.start()
                    new_agr.append(dr)
                    if agl_d is not None:
                        agl_d[cc].wait_recv()
                    dl = remote(
                        xg.at[pl.ds(orow(m16(my + h)), SQ), pl.ds(cc * HD, HD)],
                        xg.at[pl.ds(orow(m16(my + h)), SQ), pl.ds(cc * HD, HD)],
                        agl_send.at[h, cc], agl_recv.at[h, cc], left)
                    dl.start()
                    new_agl.append(dl)
            elif h == N_DEV - 1:
                for cc in range(2):
                    agr_d[cc].wait_recv()
                    agl_d[cc].wait_recv()
                new_agr = new_agl = None

            if h >= 2:
                t = h - 2
                new_rsr, new_rsl = [], []
                for cc in range(2):
                    src_r = (
                        part.at[pl.ds(erow(m16(my - 1)), SQ), pl.ds(cc * HD, HD)]
                        if t == 0 else rsb_r.at[t - 1, :, pl.ds(cc * HD, HD)])
                    dr = remote(src_r, rsb_r.at[t, :, pl.ds(cc * HD, HD)],
                                rsr_send.at[t, cc], rsr_recv.at[t, cc], right)
                    dr.start()
                    new_rsr.append(dr)
                    src_l = (
                        part.at[pl.ds(orow(m16(my + 1)), SQ), pl.ds(cc * HD, HD)]
                        if t == 0 else rsb_l.at[t - 1, :, pl.ds(cc * HD, HD)])
                    dl = remote(src_l, rsb_l.at[t, :, pl.ds(cc * HD, HD)],
                                rsl_send.at[t, cc], rsl_recv.at[t, cc], left)
                    dl.start()
                    new_rsl.append(dl)

            if h < N_DEV:
                compute_partial(erow(m16(my - h)), orow(m16(my + h)))

            if h >= 2:
                t = h - 2
                er = erow(m16(my - 2 - t))
                orr = orow(m16(my + 2 + t))
                for cc in range(2):
                    sl = slice(cc * HD, (cc + 1) * HD)
                    new_rsr[cc].wait_recv()
                    rsb_r[t, :, sl] = (rsb_r[t, :, sl]
                                       + part[pl.ds(er, SQ), sl])
                    new_rsl[cc].wait_recv()
                    rsb_l[t, :, sl] = (rsb_l[t, :, sl]
                                       + part[pl.ds(orr, SQ), sl])
                rsr_d, rsl_d = new_rsr, new_rsl

            if h < N_DEV - 1:
                for cc in range(2):
                    new_agr[cc].wait_send()
                    new_agl[cc].wait_send()
                agr_d, agl_d = new_agr, new_agl
            if h >= 2:
                for cc in range(2):
                    new_rsr[cc].wait_send()
                    new_rsl[cc].wait_send()

        out_ref[0, :, :] = rsb_r[N_DEV - 2]
        out_ref[1, :, :] = rsb_l[N_DEV - 2]

    return pl.pallas_call(
        body,
        out_shape=jax.ShapeDtypeStruct((B_LOC, SQ, D), jnp.float32),
        in_specs=[pl.BlockSpec(memory_space=pltpu.VMEM)] * 5,
        out_specs=pl.BlockSpec(memory_space=pltpu.VMEM),
        scratch_shapes=[
            pltpu.VMEM((ROWS, D), jnp.float32),
            pltpu.VMEM((ROWS, D), jnp.float32),
            pltpu.VMEM((N_DEV - 1, SQ, D), jnp.float32),
            pltpu.VMEM((N_DEV - 1, SQ, D), jnp.float32),
            pltpu.SemaphoreType.DMA((N_DEV - 1, 2)),
            pltpu.SemaphoreType.DMA((N_DEV - 1, 2)),
            pltpu.SemaphoreType.DMA((N_DEV - 1, 2)),
            pltpu.SemaphoreType.DMA((N_DEV - 1, 2)),
            pltpu.SemaphoreType.DMA((N_DEV - 1, 2)),
            pltpu.SemaphoreType.DMA((N_DEV - 1, 2)),
            pltpu.SemaphoreType.DMA((N_DEV - 1, 2)),
            pltpu.SemaphoreType.DMA((N_DEV - 1, 2)),
        ],
        compiler_params=pltpu.CompilerParams(
            collective_id=0,
            vmem_limit_bytes=63 * 1024 * 1024,
        ),
    )(x, Wq, Wk, Wv, Wo)


# device time: 124176 ns/iter; 2.3179x vs baseline; 1.1839x over previous
import math

import jax
import jax.numpy as jnp
from jax import lax
from jax.experimental import pallas as pl
from jax.experimental.pallas import tpu as pltpu

N_DEV = 16
B_LOC = 2
SQ = 128
D = 512
H_LOC = 4
DH = 64
B_GLB = N_DEV * B_LOC
ROWS = B_GLB * SQ
DH_LOC = H_LOC * DH

RING = [0, 4, 8, 12, 15, 11, 7, 3, 2, 6, 10, 14, 13, 9, 5, 1]
RING_INV = [RING.index(p) for p in range(N_DEV)]


def kernel(x, Wq, Wk, Wv, Wo):
    def body(x_ref, wq_ref, wk_ref, wv_ref, wo_ref, out_ref,
             xg, part, rsb_r, rsb_l,
             agr_send, agr_recv, agl_send, agl_recv,
             rsr_send, rsr_recv, rsl_send, rsl_recv):
        my = lax.axis_index("i")

        def m16(v):
            return lax.rem(v + 2 * N_DEV, N_DEV)

        def lut(table, idx):
            acc = jnp.int32(0)
            for j, v in enumerate(table):
                acc = acc + jnp.where(idx == j, jnp.int32(v), jnp.int32(0))
            return acc

        my_r = lut(RING_INV, my)
        left = lut(RING, m16(my_r - 1))
        right = lut(RING, m16(my_r + 1))

        def erow(c):
            return c * (2 * SQ)

        def orow(c):
            return c * (2 * SQ) + SQ

        barrier = pltpu.get_barrier_semaphore()
        for nbr in (left, right):
            pl.semaphore_signal(barrier, inc=1, device_id=(nbr,),
                                device_id_type=pl.DeviceIdType.MESH)
        pl.semaphore_wait(barrier, 2)

        xg[pl.ds(erow(my), SQ), :] = x_ref[0, :, :]
        xg[pl.ds(orow(my), SQ), :] = x_ref[1, :, :]

        rows_i = lax.broadcasted_iota(jnp.int32, (SQ, DH_LOC), 0)
        cols_i = lax.broadcasted_iota(jnp.int32, (SQ, DH_LOC), 1)
        k2 = ((jnp.remainder(cols_i, DH) // 2) * 2).astype(jnp.float32)
        inv = jnp.exp(k2 * (-math.log(10000.0) / DH))
        ang = rows_i.astype(jnp.float32) * inv
        cosv = jnp.cos(ang)
        sinv = jnp.sin(ang)

        ri = lax.broadcasted_iota(jnp.int32, (DH_LOC, DH_LOC), 0)
        ci = lax.broadcasted_iota(jnp.int32, (DH_LOC, DH_LOC), 1)
        even_c = jnp.remainder(ci, 2) == 0
        rot_m = (jnp.where(even_c & (ri == ci + 1), -1.0, 0.0)
                 + jnp.where((~even_c) & (ri == ci - 1), 1.0, 0.0)
                 ).astype(jnp.float32)

        cos2 = jnp.concatenate([cosv, cosv], axis=0)
        sin2 = jnp.concatenate([sinv, sinv], axis=0)

        wq = wq_ref[...]
        wk = wk_ref[...]
        wv = wv_ref[...]
        wo = wo_ref[...]

        def compute_partial(row_e, row_o):
            xb = jnp.concatenate(
                [xg[pl.ds(row_e, SQ), :], xg[pl.ds(row_o, SQ), :]], axis=0)
            q = jnp.dot(xb, wq, preferred_element_type=jnp.float32)
            k = jnp.dot(xb, wk, preferred_element_type=jnp.float32)
            v = jnp.dot(xb, wv, preferred_element_type=jnp.float32)
            q = q * cos2 + jnp.dot(q, rot_m,
                                   preferred_element_type=jnp.float32) * sin2
            k = k * cos2 + jnp.dot(k, rot_m,
                                   preferred_element_type=jnp.float32) * sin2
            halves = []
            for bb in range(2):
                ctx_heads = []
                for hh in range(H_LOC):
                    qs = q[bb * SQ:(bb + 1) * SQ, hh * DH:(hh + 1) * DH]
                    ks = k[bb * SQ:(bb + 1) * SQ, hh * DH:(hh + 1) * DH]
                    vs = v[bb * SQ:(bb + 1) * SQ, hh * DH:(hh + 1) * DH]
                    s = lax.dot_general(
                        qs, ks, (((1,), (1,)), ((), ())),
                        preferred_element_type=jnp.float32) * 0.125
                    mx = jnp.max(s, axis=-1, keepdims=True)
                    w = jnp.exp(s - mx)
                    w = w / jnp.sum(w, axis=-1, keepdims=True)
                    ctx_heads.append(jnp.dot(
                        w, vs, preferred_element_type=jnp.float32))
                halves.append(jnp.concatenate(ctx_heads, axis=1))
            ctx = jnp.concatenate(halves, axis=0)
            po = jnp.dot(ctx, wo, preferred_element_type=jnp.float32)
            part[pl.ds(row_e, SQ), :] = po[0:SQ, :]
            part[pl.ds(row_o, SQ), :] = po[SQ:2 * SQ, :]

        def remote(src, dst, ssem, rsem, dev):
            return pltpu.make_async_remote_copy(
                src_ref=src, dst_ref=dst, send_sem=ssem, recv_sem=rsem,
                device_id=(dev,), device_id_type=pl.DeviceIdType.MESH)

        HD = D // 2

        agr_d = agl_d = None
        for h in range(N_DEV + 1):
            if h < N_DEV:
                re_h = erow(lut(RING, m16(my_r - h)))
                ro_h = orow(lut(RING, m16(my_r + h)))

            if h < N_DEV - 1:
                new_agr, new_agl = [], []
                for cc in range(2):
                    if agr_d is not None:
                        agr_d[cc].wait_recv()
                    dr = remote(
                        xg.at[pl.ds(re_h, SQ), pl.ds(cc * HD, HD)],
                        xg.at[pl.ds(re_h, SQ), pl.ds(cc * HD, HD)],
                        agr_send.at[h, cc], agr_recv.at[h, cc], right)
                    dr.start()
                    new_agr.append(dr)
                    if agl_d is not None:
                        agl_d[cc].wait_recv()
                    dl = remote(
                        xg.at[pl.ds(ro_h, SQ), pl.ds(cc * HD, HD)],
                        xg.at[pl.ds(ro_h, SQ), pl.ds(cc * HD, HD)],
                        agl_send.at[h, cc], agl_recv.at[h, cc], left)
                    dl.start()
                    new_agl.append(dl)
            elif h == N_DEV - 1:
                for cc in range(2):
                    agr_d[cc].wait_recv()
                    agl_d[cc].wait_recv()
                new_agr = new_agl = None

            if h >= 2:
                t = h - 2
                new_rsr, new_rsl = [], []
                for cc in range(2):
                    src_r = (
                        part.at[pl.ds(erow(lut(RING, m16(my_r - 1))), SQ),
                                pl.ds(cc * HD, HD)]
                        if t == 0 else rsb_r.at[t - 1, :, pl.ds(cc * HD, HD)])
                    dr = remote(src_r, rsb_r.at[t, :, pl.ds(cc * HD, HD)],
                                rsr_send.at[t, cc], rsr_recv.at[t, cc], right)
                    dr.start()
                    new_rsr.append(dr)
                    src_l = (
                        part.at[pl.ds(orow(lut(RING, m16(my_r + 1))), SQ),
                                pl.ds(cc * HD, HD)]
                        if t == 0 else rsb_l.at[t - 1, :, pl.ds(cc * HD, HD)])
                    dl = remote(src_l, rsb_l.at[t, :, pl.ds(cc * HD, HD)],
                                rsl_send.at[t, cc], rsl_recv.at[t, cc], left)
                    dl.start()
                    new_rsl.append(dl)

            if h < N_DEV:
                compute_partial(re_h, ro_h)

            if h >= 2:
                t = h - 2
                er = erow(lut(RING, m16(my_r - 2 - t)))
                orr = orow(lut(RING, m16(my_r + 2 + t)))
                for cc in range(2):
                    sl = slice(cc * HD, (cc + 1) * HD)
                    new_rsr[cc].wait_recv()
                    rsb_r[t, :, sl] = (rsb_r[t, :, sl]
                                       + part[pl.ds(er, SQ), sl])
                    new_rsl[cc].wait_recv()
                    rsb_l[t, :, sl] = (rsb_l[t, :, sl]
                                       + part[pl.ds(orr, SQ), sl])

            if h < N_DEV - 1:
                for cc in range(2):
                    new_agr[cc].wait_send()
                    new_agl[cc].wait_send()
                agr_d, agl_d = new_agr, new_agl
            if h >= 2:
                for cc in range(2):
                    new_rsr[cc].wait_send()
                    new_rsl[cc].wait_send()

        out_ref[0, :, :] = rsb_r[N_DEV - 2]
        out_ref[1, :, :] = rsb_l[N_DEV - 2]

    return pl.pallas_call(
        body,
        out_shape=jax.ShapeDtypeStruct((B_LOC, SQ, D), jnp.float32),
        in_specs=[pl.BlockSpec(memory_space=pltpu.VMEM)] * 5,
        out_specs=pl.BlockSpec(memory_space=pltpu.VMEM),
        scratch_shapes=[
            pltpu.VMEM((ROWS, D), jnp.float32),
            pltpu.VMEM((ROWS, D), jnp.float32),
            pltpu.VMEM((N_DEV - 1, SQ, D), jnp.float32),
            pltpu.VMEM((N_DEV - 1, SQ, D), jnp.float32),
            pltpu.SemaphoreType.DMA((N_DEV - 1, 2)),
            pltpu.SemaphoreType.DMA((N_DEV - 1, 2)),
            pltpu.SemaphoreType.DMA((N_DEV - 1, 2)),
            pltpu.SemaphoreType.DMA((N_DEV - 1, 2)),
            pltpu.SemaphoreType.DMA((N_DEV - 1, 2)),
            pltpu.SemaphoreType.DMA((N_DEV - 1, 2)),
            pltpu.SemaphoreType.DMA((N_DEV - 1, 2)),
            pltpu.SemaphoreType.DMA((N_DEV - 1, 2)),
        ],
        compiler_params=pltpu.CompilerParams(
            collective_id=0,
            vmem_limit_bytes=63 * 1024 * 1024,
        ),
    )(x, Wq, Wk, Wv, Wo)


# device time: 84062 ns/iter; 3.4240x vs baseline; 1.4772x over previous
import math

import jax
import jax.numpy as jnp
from jax import lax
from jax.experimental import pallas as pl
from jax.experimental.pallas import tpu as pltpu

N_DEV = 16
B_LOC = 2
SQ = 128
D = 512
H_LOC = 4
DH = 64
B_GLB = N_DEV * B_LOC
ROWS = B_GLB * SQ
DH_LOC = H_LOC * DH

RING = [0, 4, 8, 12, 15, 11, 7, 3, 2, 6, 10, 14, 13, 9, 5, 1]
RING_INV = [RING.index(p) for p in range(N_DEV)]


def kernel(x, Wq, Wk, Wv, Wo):
    def body(x_ref, wq_ref, wk_ref, wv_ref, wo_ref, out_ref,
             xg, part, rsb_r, rsb_l,
             agr_send, agr_recv, agl_send, agl_recv,
             rsr_send, rsr_recv, rsl_send, rsl_recv):
        my = lax.axis_index("i")

        def m16(v):
            return lax.rem(v + 2 * N_DEV, N_DEV)

        def lut(table, idx):
            acc = jnp.int32(0)
            for j, v in enumerate(table):
                acc = acc + jnp.where(idx == j, jnp.int32(v), jnp.int32(0))
            return acc

        my_r = lut(RING_INV, my)
        left = lut(RING, m16(my_r - 1))
        right = lut(RING, m16(my_r + 1))

        def erow(c):
            return c * (2 * SQ)

        def orow(c):
            return c * (2 * SQ) + SQ

        barrier = pltpu.get_barrier_semaphore()
        for nbr in (left, right):
            pl.semaphore_signal(barrier, inc=1, device_id=(nbr,),
                                device_id_type=pl.DeviceIdType.MESH)
        pl.semaphore_wait(barrier, 2)

        xg[pl.ds(erow(my), SQ), :] = x_ref[0, :, :].astype(jnp.bfloat16)
        xg[pl.ds(orow(my), SQ), :] = x_ref[1, :, :].astype(jnp.bfloat16)

        rows_i = lax.broadcasted_iota(jnp.int32, (SQ, DH_LOC), 0)
        cols_i = lax.broadcasted_iota(jnp.int32, (SQ, DH_LOC), 1)
        k2 = ((jnp.remainder(cols_i, DH) // 2) * 2).astype(jnp.float32)
        inv = jnp.exp(k2 * (-math.log(10000.0) / DH))
        ang = rows_i.astype(jnp.float32) * inv
        cosv = jnp.cos(ang)
        sinv = jnp.sin(ang)

        ri = lax.broadcasted_iota(jnp.int32, (DH_LOC, DH_LOC), 0)
        ci = lax.broadcasted_iota(jnp.int32, (DH_LOC, DH_LOC), 1)
        even_c = jnp.remainder(ci, 2) == 0
        rot_m = (jnp.where(even_c & (ri == ci + 1), -1.0, 0.0)
                 + jnp.where((~even_c) & (ri == ci - 1), 1.0, 0.0)
                 ).astype(jnp.float32)

        cos2 = jnp.concatenate([cosv, cosv], axis=0)
        sin2 = jnp.concatenate([sinv, sinv], axis=0)

        wq = wq_ref[...]
        wk = wk_ref[...]
        wv = wv_ref[...]
        wo = wo_ref[...]

        def compute_partial(row_e, row_o):
            xb = jnp.concatenate(
                [xg[pl.ds(row_e, SQ), :], xg[pl.ds(row_o, SQ), :]],
                axis=0).astype(jnp.float32)
            q = jnp.dot(xb, wq, preferred_element_type=jnp.float32)
            k = jnp.dot(xb, wk, preferred_element_type=jnp.float32)
            v = jnp.dot(xb, wv, preferred_element_type=jnp.float32)
            q = q * cos2 + jnp.dot(q, rot_m,
                                   preferred_element_type=jnp.float32) * sin2
            k = k * cos2 + jnp.dot(k, rot_m,
                                   preferred_element_type=jnp.float32) * sin2
            halves = []
            for bb in range(2):
                ctx_heads = []
                for hh in range(H_LOC):
                    qs = q[bb * SQ:(bb + 1) * SQ, hh * DH:(hh + 1) * DH]
                    ks = k[bb * SQ:(bb + 1) * SQ, hh * DH:(hh + 1) * DH]
                    vs = v[bb * SQ:(bb + 1) * SQ, hh * DH:(hh + 1) * DH]
                    s = lax.dot_general(
                        qs, ks, (((1,), (1,)), ((), ())),
                        preferred_element_type=jnp.float32) * 0.125
                    mx = jnp.max(s, axis=-1, keepdims=True)
                    w = jnp.exp(s - mx)
                    w = w / jnp.sum(w, axis=-1, keepdims=True)
                    ctx_heads.append(jnp.dot(
                        w, vs, preferred_element_type=jnp.float32))
                halves.append(jnp.concatenate(ctx_heads, axis=1))
            ctx = jnp.concatenate(halves, axis=0)
            po = jnp.dot(ctx, wo,
                         preferred_element_type=jnp.float32
                         ).astype(jnp.bfloat16)
            part[pl.ds(row_e, SQ), :] = po[0:SQ, :]
            part[pl.ds(row_o, SQ), :] = po[SQ:2 * SQ, :]

        def remote(src, dst, ssem, rsem, dev):
            return pltpu.make_async_remote_copy(
                src_ref=src, dst_ref=dst, send_sem=ssem, recv_sem=rsem,
                device_id=(dev,), device_id_type=pl.DeviceIdType.MESH)

        HD = D // 2

        agr_d = agl_d = None
        for h in range(N_DEV + 1):
            if h < N_DEV:
                re_h = erow(lut(RING, m16(my_r - h)))
                ro_h = orow(lut(RING, m16(my_r + h)))

            if h < N_DEV - 1:
                new_agr, new_agl = [], []
                for cc in range(2):
                    if agr_d is not None:
                        agr_d[cc].wait_recv()
                    dr = remote(
                        xg.at[pl.ds(re_h, SQ), pl.ds(cc * HD, HD)],
                        xg.at[pl.ds(re_h, SQ), pl.ds(cc * HD, HD)],
                        agr_send.at[h, cc], agr_recv.at[h, cc], right)
                    dr.start()
                    new_agr.append(dr)
                    if agl_d is not None:
                        agl_d[cc].wait_recv()
                    dl = remote(
                        xg.at[pl.ds(ro_h, SQ), pl.ds(cc * HD, HD)],
                        xg.at[pl.ds(ro_h, SQ), pl.ds(cc * HD, HD)],
                        agl_send.at[h, cc], agl_recv.at[h, cc], left)
                    dl.start()
                    new_agl.append(dl)
            elif h == N_DEV - 1:
                for cc in range(2):
                    agr_d[cc].wait_recv()
                    agl_d[cc].wait_recv()
                new_agr = new_agl = None

            if h >= 2:
                t = h - 2
                new_rsr, new_rsl = [], []
                for cc in range(2):
                    src_r = (
                        part.at[pl.ds(erow(lut(RING, m16(my_r - 1))), SQ),
                                pl.ds(cc * HD, HD)]
                        if t == 0 else rsb_r.at[t - 1, :, pl.ds(cc * HD, HD)])
                    dr = remote(src_r, rsb_r.at[t, :, pl.ds(cc * HD, HD)],
                                rsr_send.at[t, cc], rsr_recv.at[t, cc], right)
                    dr.start()
                    new_rsr.append(dr)
                    src_l = (
                        part.at[pl.ds(orow(lut(RING, m16(my_r + 1))), SQ),
                                pl.ds(cc * HD, HD)]
                        if t == 0 else rsb_l.at[t - 1, :, pl.ds(cc * HD, HD)])
                    dl = remote(src_l, rsb_l.at[t, :, pl.ds(cc * HD, HD)],
                                rsl_send.at[t, cc], rsl_recv.at[t, cc], left)
                    dl.start()
                    new_rsl.append(dl)

            if h < N_DEV:
                compute_partial(re_h, ro_h)

            if h >= 2:
                t = h - 2
                er = erow(lut(RING, m16(my_r - 2 - t)))
                orr = orow(lut(RING, m16(my_r + 2 + t)))
                for cc in range(2):
                    sl = slice(cc * HD, (cc + 1) * HD)
                    new_rsr[cc].wait_recv()
                    rsb_r[t, :, sl] = (
                        rsb_r[t, :, sl].astype(jnp.float32)
                        + part[pl.ds(er, SQ), sl].astype(jnp.float32)
                    ).astype(jnp.bfloat16)
                    new_rsl[cc].wait_recv()
                    rsb_l[t, :, sl] = (
                        rsb_l[t, :, sl].astype(jnp.float32)
                        + part[pl.ds(orr, SQ), sl].astype(jnp.float32)
                    ).astype(jnp.bfloat16)

            if h < N_DEV - 1:
                for cc in range(2):
                    new_agr[cc].wait_send()
                    new_agl[cc].wait_send()
                agr_d, agl_d = new_agr, new_agl
            if h >= 2:
                for cc in range(2):
                    new_rsr[cc].wait_send()
                    new_rsl[cc].wait_send()

        out_ref[0, :, :] = rsb_r[N_DEV - 2].astype(jnp.float32)
        out_ref[1, :, :] = rsb_l[N_DEV - 2].astype(jnp.float32)

    return pl.pallas_call(
        body,
        out_shape=jax.ShapeDtypeStruct((B_LOC, SQ, D), jnp.float32),
        in_specs=[pl.BlockSpec(memory_space=pltpu.VMEM)] * 5,
        out_specs=pl.BlockSpec(memory_space=pltpu.VMEM),
        scratch_shapes=[
            pltpu.VMEM((ROWS, D), jnp.bfloat16),
            pltpu.VMEM((ROWS, D), jnp.bfloat16),
            pltpu.VMEM((N_DEV - 1, SQ, D), jnp.bfloat16),
            pltpu.VMEM((N_DEV - 1, SQ, D), jnp.bfloat16),
            pltpu.SemaphoreType.DMA((N_DEV - 1, 2)),
            pltpu.SemaphoreType.DMA((N_DEV - 1, 2)),
            pltpu.SemaphoreType.DMA((N_DEV - 1, 2)),
            pltpu.SemaphoreType.DMA((N_DEV - 1, 2)),
            pltpu.SemaphoreType.DMA((N_DEV - 1, 2)),
            pltpu.SemaphoreType.DMA((N_DEV - 1, 2)),
            pltpu.SemaphoreType.DMA((N_DEV - 1, 2)),
            pltpu.SemaphoreType.DMA((N_DEV - 1, 2)),
        ],
        compiler_params=pltpu.CompilerParams(
            collective_id=0,
            vmem_limit_bytes=63 * 1024 * 1024,
        ),
    )(x, Wq, Wk, Wv, Wo)


# device time: 83256 ns/iter; 3.4571x vs baseline; 1.0097x over previous
import math

import jax
import jax.numpy as jnp
from jax import lax
from jax.experimental import pallas as pl
from jax.experimental.pallas import tpu as pltpu

N_DEV = 16
B_LOC = 2
SQ = 128
D = 512
H_LOC = 4
DH = 64
B_GLB = N_DEV * B_LOC
ROWS = B_GLB * SQ
DH_LOC = H_LOC * DH

RING = [0, 4, 8, 12, 15, 11, 7, 3, 2, 6, 10, 14, 13, 9, 5, 1]
RING_INV = [RING.index(p) for p in range(N_DEV)]


def kernel(x, Wq, Wk, Wv, Wo):
    def body(x_ref, wq_ref, wk_ref, wv_ref, wo_ref, out_ref,
             xg, part, rsb_r, rsb_l,
             agr_send, agr_recv, agl_send, agl_recv,
             rsr_send, rsr_recv, rsl_send, rsl_recv):
        my = lax.axis_index("i")

        def m16(v):
            return lax.rem(v + 2 * N_DEV, N_DEV)

        def lut(table, idx):
            acc = jnp.int32(0)
            for j, v in enumerate(table):
                acc = acc + jnp.where(idx == j, jnp.int32(v), jnp.int32(0))
            return acc

        my_r = lut(RING_INV, my)
        left = lut(RING, m16(my_r - 1))
        right = lut(RING, m16(my_r + 1))

        def erow(c):
            return c * (2 * SQ)

        def orow(c):
            return c * (2 * SQ) + SQ

        barrier = pltpu.get_barrier_semaphore()
        for nbr in (left, right):
            pl.semaphore_signal(barrier, inc=1, device_id=(nbr,),
                                device_id_type=pl.DeviceIdType.MESH)
        pl.semaphore_wait(barrier, 2)

        xg[pl.ds(erow(my), SQ), :] = x_ref[0, :, :].astype(jnp.bfloat16)
        xg[pl.ds(orow(my), SQ), :] = x_ref[1, :, :].astype(jnp.bfloat16)

        rows_i = lax.broadcasted_iota(jnp.int32, (SQ, DH_LOC), 0)
        cols_i = lax.broadcasted_iota(jnp.int32, (SQ, DH_LOC), 1)
        k2 = ((jnp.remainder(cols_i, DH) // 2) * 2).astype(jnp.float32)
        inv = jnp.exp(k2 * (-math.log(10000.0) / DH))
        ang = rows_i.astype(jnp.float32) * inv
        cosv = jnp.cos(ang)
        sinv = jnp.sin(ang)

        ri = lax.broadcasted_iota(jnp.int32, (DH_LOC, DH_LOC), 0)
        ci = lax.broadcasted_iota(jnp.int32, (DH_LOC, DH_LOC), 1)
        even_c = jnp.remainder(ci, 2) == 0
        rot_m = (jnp.where(even_c & (ri == ci + 1), -1.0, 0.0)
                 + jnp.where((~even_c) & (ri == ci - 1), 1.0, 0.0)
                 ).astype(jnp.float32)

        cos2 = jnp.concatenate([cosv, cosv], axis=0)
        sin2 = jnp.concatenate([sinv, sinv], axis=0)

        wq = wq_ref[...].astype(jnp.bfloat16)
        wk = wk_ref[...].astype(jnp.bfloat16)
        wv = wv_ref[...].astype(jnp.bfloat16)
        wo = wo_ref[...].astype(jnp.bfloat16)

        def compute_partial(row_e, row_o):
            xb = jnp.concatenate(
                [xg[pl.ds(row_e, SQ), :], xg[pl.ds(row_o, SQ), :]], axis=0)
            q = jnp.dot(xb, wq, preferred_element_type=jnp.float32)
            k = jnp.dot(xb, wk, preferred_element_type=jnp.float32)
            v = jnp.dot(xb, wv, preferred_element_type=jnp.float32)
            q = q * cos2 + jnp.dot(q, rot_m,
                                   preferred_element_type=jnp.float32) * sin2
            k = k * cos2 + jnp.dot(k, rot_m,
                                   preferred_element_type=jnp.float32) * sin2
            halves = []
            for bb in range(2):
                ctx_heads = []
                for hh in range(H_LOC):
                    qs = q[bb * SQ:(bb + 1) * SQ, hh * DH:(hh + 1) * DH]
                    ks = k[bb * SQ:(bb + 1) * SQ, hh * DH:(hh + 1) * DH]
                    vs = v[bb * SQ:(bb + 1) * SQ, hh * DH:(hh + 1) * DH]
                    s = lax.dot_general(
                        qs, ks, (((1,), (1,)), ((), ())),
                        preferred_element_type=jnp.float32) * 0.125
                    mx = jnp.max(s, axis=-1, keepdims=True)
                    w = jnp.exp(s - mx)
                    w = w / jnp.sum(w, axis=-1, keepdims=True)
                    ctx_heads.append(jnp.dot(
                        w, vs, preferred_element_type=jnp.float32))
                halves.append(jnp.concatenate(ctx_heads, axis=1))
            ctx = jnp.concatenate(halves, axis=0).astype(jnp.bfloat16)
            po = jnp.dot(ctx, wo,
                         preferred_element_type=jnp.float32
                         ).astype(jnp.bfloat16)
            part[pl.ds(row_e, SQ), :] = po[0:SQ, :]
            part[pl.ds(row_o, SQ), :] = po[SQ:2 * SQ, :]

        def remote(src, dst, ssem, rsem, dev):
            return pltpu.make_async_remote_copy(
                src_ref=src, dst_ref=dst, send_sem=ssem, recv_sem=rsem,
                device_id=(dev,), device_id_type=pl.DeviceIdType.MESH)

        HD = D // 2

        agr_d = agl_d = None
        for h in range(N_DEV + 1):
            if h < N_DEV:
                re_h = erow(lut(RING, m16(my_r - h)))
                ro_h = orow(lut(RING, m16(my_r + h)))

            if h < N_DEV - 1:
                if agr_d is not None:
                    agr_d.wait_recv()
                new_agr = remote(xg.at[pl.ds(re_h, SQ)],
                                 xg.at[pl.ds(re_h, SQ)],
                                 agr_send.at[h], agr_recv.at[h], right)
                new_agr.start()
                if agl_d is not None:
                    agl_d.wait_recv()
                new_agl = remote(xg.at[pl.ds(ro_h, SQ)],
                                 xg.at[pl.ds(ro_h, SQ)],
                                 agl_send.at[h], agl_recv.at[h], left)
                new_agl.start()
            elif h == N_DEV - 1:
                agr_d.wait_recv()
                agl_d.wait_recv()
                new_agr = new_agl = None

            if h >= 2:
                t = h - 2
                src_r = (part.at[pl.ds(erow(lut(RING, m16(my_r - 1))), SQ)]
                         if t == 0 else rsb_r.at[t - 1])
                new_rsr = remote(src_r, rsb_r.at[t],
                                 rsr_send.at[t], rsr_recv.at[t], right)
                new_rsr.start()
                src_l = (part.at[pl.ds(orow(lut(RING, m16(my_r + 1))), SQ)]
                         if t == 0 else rsb_l.at[t - 1])
                new_rsl = remote(src_l, rsb_l.at[t],
                                 rsl_send.at[t], rsl_recv.at[t], left)
                new_rsl.start()

            if h < N_DEV:
                compute_partial(re_h, ro_h)

            if h >= 2:
                t = h - 2
                er = erow(lut(RING, m16(my_r - 2 - t)))
                orr = orow(lut(RING, m16(my_r + 2 + t)))
                new_rsr.wait_recv()
                rsb_r[t] = (rsb_r[t].astype(jnp.float32)
                            + part[pl.ds(er, SQ), :].astype(jnp.float32)
                            ).astype(jnp.bfloat16)
                new_rsl.wait_recv()
                rsb_l[t] = (rsb_l[t].astype(jnp.float32)
                            + part[pl.ds(orr, SQ), :].astype(jnp.float32)
                            ).astype(jnp.bfloat16)

            if h < N_DEV - 1:
                new_agr.wait_send()
                new_agl.wait_send()
                agr_d, agl_d = new_agr, new_agl
            if h >= 2:
                new_rsr.wait_send()
                new_rsl.wait_send()

        out_ref[0, :, :] = rsb_r[N_DEV - 2].astype(jnp.float32)
        out_ref[1, :, :] = rsb_l[N_DEV - 2].astype(jnp.float32)

    return pl.pallas_call(
        body,
        out_shape=jax.ShapeDtypeStruct((B_LOC, SQ, D), jnp.float32),
        in_specs=[pl.BlockSpec(memory_space=pltpu.VMEM)] * 5,
        out_specs=pl.BlockSpec(memory_space=pltpu.VMEM),
        scratch_shapes=[
            pltpu.VMEM((ROWS, D), jnp.bfloat16),
            pltpu.VMEM((ROWS, D), jnp.bfloat16),
            pltpu.VMEM((N_DEV - 1, SQ, D), jnp.bfloat16),
            pltpu.VMEM((N_DEV - 1, SQ, D), jnp.bfloat16),
            pltpu.SemaphoreType.DMA((N_DEV - 1,)),
            pltpu.SemaphoreType.DMA((N_DEV - 1,)),
            pltpu.SemaphoreType.DMA((N_DEV - 1,)),
            pltpu.SemaphoreType.DMA((N_DEV - 1,)),
            pltpu.SemaphoreType.DMA((N_DEV - 1,)),
            pltpu.SemaphoreType.DMA((N_DEV - 1,)),
            pltpu.SemaphoreType.DMA((N_DEV - 1,)),
            pltpu.SemaphoreType.DMA((N_DEV - 1,)),
        ],
        compiler_params=pltpu.CompilerParams(
            collective_id=0,
            vmem_limit_bytes=63 * 1024 * 1024,
        ),
    )(x, Wq, Wk, Wv, Wo)


# device time: 83024 ns/iter; 3.4668x vs baseline; 1.0028x over previous
import math

import jax
import jax.numpy as jnp
from jax import lax
from jax.experimental import pallas as pl
from jax.experimental.pallas import tpu as pltpu

N_DEV = 16
B_LOC = 2
SQ = 128
D = 512
H_LOC = 4
DH = 64
B_GLB = N_DEV * B_LOC
ROWS = B_GLB * SQ
DH_LOC = H_LOC * DH

RING = [0, 4, 8, 12, 15, 11, 7, 3, 2, 6, 10, 14, 13, 9, 5, 1]
RING_INV = [RING.index(p) for p in range(N_DEV)]


def kernel(x, Wq, Wk, Wv, Wo):
    def body(x_ref, wq_ref, wk_ref, wv_ref, wo_ref, out_ref,
             xg, part, rsb_r, rsb_l,
             agr_send, agr_recv, agl_send, agl_recv,
             rsr_send, rsr_recv, rsl_send, rsl_recv):
        my = lax.axis_index("i")

        def m16(v):
            return lax.rem(v + 2 * N_DEV, N_DEV)

        def lut(table, idx):
            acc = jnp.int32(0)
            for j, v in enumerate(table):
                acc = acc + jnp.where(idx == j, jnp.int32(v), jnp.int32(0))
            return acc

        my_r = lut(RING_INV, my)
        left = lut(RING, m16(my_r - 1))
        right = lut(RING, m16(my_r + 1))

        def erow(c):
            return c * (2 * SQ)

        def orow(c):
            return c * (2 * SQ) + SQ

        barrier = pltpu.get_barrier_semaphore()
        for nbr in (left, right):
            pl.semaphore_signal(barrier, inc=1, device_id=(nbr,),
                                device_id_type=pl.DeviceIdType.MESH)
        pl.semaphore_wait(barrier, 2)

        xg[pl.ds(erow(my), SQ), :] = x_ref[0, :, :].astype(jnp.bfloat16)
        xg[pl.ds(orow(my), SQ), :] = x_ref[1, :, :].astype(jnp.bfloat16)

        rows_i = lax.broadcasted_iota(jnp.int32, (SQ, DH_LOC), 0)
        cols_i = lax.broadcasted_iota(jnp.int32, (SQ, DH_LOC), 1)
        k2 = ((jnp.remainder(cols_i, DH) // 2) * 2).astype(jnp.float32)
        inv = jnp.exp(k2 * (-math.log(10000.0) / DH))
        ang = rows_i.astype(jnp.float32) * inv
        cosv = jnp.cos(ang)
        sinv = jnp.sin(ang)

        ri = lax.broadcasted_iota(jnp.int32, (DH_LOC, DH_LOC), 0)
        ci = lax.broadcasted_iota(jnp.int32, (DH_LOC, DH_LOC), 1)
        even_c = jnp.remainder(ci, 2) == 0
        rot_m = (jnp.where(even_c & (ri == ci + 1), -1.0, 0.0)
                 + jnp.where((~even_c) & (ri == ci - 1), 1.0, 0.0)
                 ).astype(jnp.float32)

        cos2 = jnp.concatenate([cosv, cosv], axis=0)
        sin2 = jnp.concatenate([sinv, sinv], axis=0)

        wq = wq_ref[...].astype(jnp.bfloat16)
        wk = wk_ref[...].astype(jnp.bfloat16)
        wv = wv_ref[...].astype(jnp.bfloat16)
        wo = wo_ref[...].astype(jnp.bfloat16)

        def compute_partial(row_e, row_o):
            xb = jnp.concatenate(
                [xg[pl.ds(row_e, SQ), :], xg[pl.ds(row_o, SQ), :]], axis=0)
            q = jnp.dot(xb, wq, preferred_element_type=jnp.float32)
            k = jnp.dot(xb, wk, preferred_element_type=jnp.float32)
            v = jnp.dot(xb, wv, preferred_element_type=jnp.float32)
            q = q * cos2 + jnp.dot(q, rot_m,
                                   preferred_element_type=jnp.float32) * sin2
            k = k * cos2 + jnp.dot(k, rot_m,
                                   preferred_element_type=jnp.float32) * sin2
            halves = []
            for bb in range(2):
                ctx_heads = []
                for hh in range(H_LOC):
                    qs = q[bb * SQ:(bb + 1) * SQ, hh * DH:(hh + 1) * DH]
                    ks = k[bb * SQ:(bb + 1) * SQ, hh * DH:(hh + 1) * DH]
                    vs = v[bb * SQ:(bb + 1) * SQ, hh * DH:(hh + 1) * DH]
                    s = lax.dot_general(
                        qs, ks, (((1,), (1,)), ((), ())),
                        preferred_element_type=jnp.float32) * 0.125
                    mx = jnp.max(s, axis=-1, keepdims=True)
                    w = jnp.exp(s - mx)
                    w = w / jnp.sum(w, axis=-1, keepdims=True)
                    ctx_heads.append(jnp.dot(
                        w, vs, preferred_element_type=jnp.float32))
                halves.append(jnp.concatenate(ctx_heads, axis=1))
            ctx = jnp.concatenate(halves, axis=0).astype(jnp.bfloat16)
            return jnp.dot(ctx, wo, preferred_element_type=jnp.float32)

        def remote(src, dst, ssem, rsem, dev):
            return pltpu.make_async_remote_copy(
                src_ref=src, dst_ref=dst, send_sem=ssem, recv_sem=rsem,
                device_id=(dev,), device_id_type=pl.DeviceIdType.MESH)

        HD = D // 2

        agr_d = agl_d = None
        for h in range(N_DEV + 1):
            if h < N_DEV:
                re_h = erow(lut(RING, m16(my_r - h)))
                ro_h = orow(lut(RING, m16(my_r + h)))

            if h < N_DEV - 1:
                if agr_d is not None:
                    agr_d.wait_recv()
                new_agr = remote(xg.at[pl.ds(re_h, SQ)],
                                 xg.at[pl.ds(re_h, SQ)],
                                 agr_send.at[h], agr_recv.at[h], right)
                new_agr.start()
                if agl_d is not None:
                    agl_d.wait_recv()
                new_agl = remote(xg.at[pl.ds(ro_h, SQ)],
                                 xg.at[pl.ds(ro_h, SQ)],
                                 agl_send.at[h], agl_recv.at[h], left)
                new_agl.start()
            elif h == N_DEV - 1:
                agr_d.wait_recv()
                agl_d.wait_recv()
                new_agr = new_agl = None

            if h >= 2:
                t = h - 2
                src_r = (part.at[pl.ds(erow(lut(RING, m16(my_r - 1))), SQ)]
                         if t == 0 else rsb_r.at[t - 1])
                new_rsr = remote(src_r, rsb_r.at[t],
                                 rsr_send.at[t], rsr_recv.at[t], right)
                new_rsr.start()
                src_l = (part.at[pl.ds(orow(lut(RING, m16(my_r + 1))), SQ)]
                         if t == 0 else rsb_l.at[t - 1])
                new_rsl = remote(src_l, rsb_l.at[t],
                                 rsl_send.at[t], rsl_recv.at[t], left)
                new_rsl.start()

            if h < N_DEV:
                po = compute_partial(re_h, ro_h)
                if h <= 1:
                    part[pl.ds(re_h, SQ), :] = po[0:SQ, :].astype(jnp.bfloat16)
                    part[pl.ds(ro_h, SQ), :] = po[SQ:2 * SQ, :].astype(
                        jnp.bfloat16)

            if h >= 2:
                t = h - 2
                if h < N_DEV:
                    add_e = po[0:SQ, :]
                    add_o = po[SQ:2 * SQ, :]
                else:
                    add_e = part[pl.ds(erow(my), SQ), :].astype(jnp.float32)
                    add_o = part[pl.ds(orow(my), SQ), :].astype(jnp.float32)
                new_rsr.wait_recv()
                rsb_r[t] = (rsb_r[t].astype(jnp.float32)
                            + add_e).astype(jnp.bfloat16)
                new_rsl.wait_recv()
                rsb_l[t] = (rsb_l[t].astype(jnp.float32)
                            + add_o).astype(jnp.bfloat16)

            if h < N_DEV - 1:
                new_agr.wait_send()
                new_agl.wait_send()
                agr_d, agl_d = new_agr, new_agl
            if h >= 2:
                new_rsr.wait_send()
                new_rsl.wait_send()

        out_ref[0, :, :] = rsb_r[N_DEV - 2].astype(jnp.float32)
        out_ref[1, :, :] = rsb_l[N_DEV - 2].astype(jnp.float32)

    return pl.pallas_call(
        body,
        out_shape=jax.ShapeDtypeStruct((B_LOC, SQ, D), jnp.float32),
        in_specs=[pl.BlockSpec(memory_space=pltpu.VMEM)] * 5,
        out_specs=pl.BlockSpec(memory_space=pltpu.VMEM),
        scratch_shapes=[
            pltpu.VMEM((ROWS, D), jnp.bfloat16),
            pltpu.VMEM((ROWS, D), jnp.bfloat16),
            pltpu.VMEM((N_DEV - 1, SQ, D), jnp.bfloat16),
            pltpu.VMEM((N_DEV - 1, SQ, D), jnp.bfloat16),
            pltpu.SemaphoreType.DMA((N_DEV - 1,)),
            pltpu.SemaphoreType.DMA((N_DEV - 1,)),
            pltpu.SemaphoreType.DMA((N_DEV - 1,)),
            pltpu.SemaphoreType.DMA((N_DEV - 1,)),
            pltpu.SemaphoreType.DMA((N_DEV - 1,)),
            pltpu.SemaphoreType.DMA((N_DEV - 1,)),
            pltpu.SemaphoreType.DMA((N_DEV - 1,)),
            pltpu.SemaphoreType.DMA((N_DEV - 1,)),
        ],
        compiler_params=pltpu.CompilerParams(
            collective_id=0,
            vmem_limit_bytes=63 * 1024 * 1024,
        ),
    )(x, Wq, Wk, Wv, Wo)


# device time: 75959 ns/iter; 3.7893x vs baseline; 1.0930x over previous
import math

import jax
import jax.numpy as jnp
from jax import lax
from jax.experimental import pallas as pl
from jax.experimental.pallas import tpu as pltpu

N_DEV = 16
B_LOC = 2
SQ = 128
D = 512
H_LOC = 4
DH = 64
B_GLB = N_DEV * B_LOC
ROWS = B_GLB * SQ
DH_LOC = H_LOC * DH

RING = [0, 4, 8, 12, 15, 11, 7, 3, 2, 6, 10, 14, 13, 9, 5, 1]
RING_INV = [RING.index(p) for p in range(N_DEV)]


def kernel(x, Wq, Wk, Wv, Wo):
    def body(x_ref, wq_ref, wk_ref, wv_ref, wo_ref, out_ref,
             xg, part, rsb_r, rsb_l,
             agr_send, agr_recv, agl_send, agl_recv,
             rsr_send, rsr_recv, rsl_send, rsl_recv):
        my = lax.axis_index("i")

        def m16(v):
            return lax.rem(v + 2 * N_DEV, N_DEV)

        def lut(table, idx):
            acc = jnp.int32(0)
            for j, v in enumerate(table):
                acc = acc + jnp.where(idx == j, jnp.int32(v), jnp.int32(0))
            return acc

        my_r = lut(RING_INV, my)
        left = lut(RING, m16(my_r - 1))
        right = lut(RING, m16(my_r + 1))

        def erow(c):
            return c * (2 * SQ)

        def orow(c):
            return c * (2 * SQ) + SQ

        barrier = pltpu.get_barrier_semaphore()
        for nbr in (left, right):
            pl.semaphore_signal(barrier, inc=1, device_id=(nbr,),
                                device_id_type=pl.DeviceIdType.MESH)
        pl.semaphore_wait(barrier, 2)

        xg[pl.ds(erow(my), SQ), :] = x_ref[0, :, :].astype(jnp.bfloat16)
        xg[pl.ds(orow(my), SQ), :] = x_ref[1, :, :].astype(jnp.bfloat16)

        rows_i = lax.broadcasted_iota(jnp.int32, (SQ, DH_LOC), 0)
        cols_i = lax.broadcasted_iota(jnp.int32, (SQ, DH_LOC), 1)
        k2 = ((jnp.remainder(cols_i, DH) // 2) * 2).astype(jnp.float32)
        inv = jnp.exp(k2 * (-math.log(10000.0) / DH))
        ang = rows_i.astype(jnp.float32) * inv
        cosv = jnp.cos(ang)
        sinv = jnp.sin(ang)

        ri = lax.broadcasted_iota(jnp.int32, (DH_LOC, DH_LOC), 0)
        ci = lax.broadcasted_iota(jnp.int32, (DH_LOC, DH_LOC), 1)
        even_c = jnp.remainder(ci, 2) == 0
        rot_m = (jnp.where(even_c & (ri == ci + 1), -1.0, 0.0)
                 + jnp.where((~even_c) & (ri == ci - 1), 1.0, 0.0)
                 ).astype(jnp.float32)

        cos2 = jnp.concatenate([cosv, cosv], axis=0)
        sin2 = jnp.concatenate([sinv, sinv], axis=0)

        wq = wq_ref[...].astype(jnp.bfloat16)
        wk = wk_ref[...].astype(jnp.bfloat16)
        wv = wv_ref[...].astype(jnp.bfloat16)
        wo = wo_ref[...].astype(jnp.bfloat16)

        def compute_partial(row_e, row_o):
            xb = jnp.concatenate(
                [xg[pl.ds(row_e, SQ), :], xg[pl.ds(row_o, SQ), :]], axis=0)
            q = jnp.dot(xb, wq, preferred_element_type=jnp.float32)
            k = jnp.dot(xb, wk, preferred_element_type=jnp.float32)
            v = jnp.dot(xb, wv, preferred_element_type=jnp.float32)
            q = q * cos2 + jnp.dot(q, rot_m,
                                   preferred_element_type=jnp.float32) * sin2
            k = k * cos2 + jnp.dot(k, rot_m,
                                   preferred_element_type=jnp.float32) * sin2
            q = q.astype(jnp.bfloat16)
            k = k.astype(jnp.bfloat16)
            v = v.astype(jnp.bfloat16)
            halves = []
            for bb in range(2):
                ctx_heads = []
                for hh in range(H_LOC):
                    qs = q[bb * SQ:(bb + 1) * SQ, hh * DH:(hh + 1) * DH]
                    ks = k[bb * SQ:(bb + 1) * SQ, hh * DH:(hh + 1) * DH]
                    vs = v[bb * SQ:(bb + 1) * SQ, hh * DH:(hh + 1) * DH]
                    s = lax.dot_general(
                        qs, ks, (((1,), (1,)), ((), ())),
                        preferred_element_type=jnp.float32) * 0.125
                    mx = jnp.max(s, axis=-1, keepdims=True)
                    w = jnp.exp(s - mx)
                    w = (w / jnp.sum(w, axis=-1, keepdims=True)
                         ).astype(jnp.bfloat16)
                    ctx_heads.append(jnp.dot(
                        w, vs,
                        preferred_element_type=jnp.float32
                    ).astype(jnp.bfloat16))
                halves.append(jnp.concatenate(ctx_heads, axis=1))
            ctx = jnp.concatenate(halves, axis=0)
            return jnp.dot(ctx, wo, preferred_element_type=jnp.float32)

        def remote(src, dst, ssem, rsem, dev):
            return pltpu.make_async_remote_copy(
                src_ref=src, dst_ref=dst, send_sem=ssem, recv_sem=rsem,
                device_id=(dev,), device_id_type=pl.DeviceIdType.MESH)

        HD = D // 2

        agr_d = agl_d = None
        for h in range(N_DEV + 1):
            if h < N_DEV:
                re_h = erow(lut(RING, m16(my_r - h)))
                ro_h = orow(lut(RING, m16(my_r + h)))

            if h >= 2:
                t = h - 2
                src_r = (part.at[pl.ds(erow(lut(RING, m16(my_r - 1))), SQ)]
                         if t == 0 else rsb_r.at[t - 1])
                new_rsr = remote(src_r, rsb_r.at[t],
                                 rsr_send.at[t], rsr_recv.at[t], right)
                new_rsr.start()
                src_l = (part.at[pl.ds(orow(lut(RING, m16(my_r + 1))), SQ)]
                         if t == 0 else rsb_l.at[t - 1])
                new_rsl = remote(src_l, rsb_l.at[t],
                                 rsl_send.at[t], rsl_recv.at[t], left)
                new_rsl.start()

            if h < N_DEV - 1:
                if agr_d is not None:
                    agr_d.wait_recv()
                new_agr = remote(xg.at[pl.ds(re_h, SQ)],
                                 xg.at[pl.ds(re_h, SQ)],
                                 agr_send.at[h], agr_recv.at[h], right)
                new_agr.start()
                if agl_d is not None:
                    agl_d.wait_recv()
                new_agl = remote(xg.at[pl.ds(ro_h, SQ)],
                                 xg.at[pl.ds(ro_h, SQ)],
                                 agl_send.at[h], agl_recv.at[h], left)
                new_agl.start()
            elif h == N_DEV - 1:
                agr_d.wait_recv()
                agl_d.wait_recv()
                new_agr = new_agl = None

            if h < N_DEV:
                po = compute_partial(re_h, ro_h)
                if h <= 1:
                    part[pl.ds(re_h, SQ), :] = po[0:SQ, :].astype(jnp.bfloat16)
                    part[pl.ds(ro_h, SQ), :] = po[SQ:2 * SQ, :].astype(
                        jnp.bfloat16)

            if h >= 2:
                t = h - 2
                if h < N_DEV:
                    add_e = po[0:SQ, :]
                    add_o = po[SQ:2 * SQ, :]
                else:
                    add_e = part[pl.ds(erow(my), SQ), :].astype(jnp.float32)
                    add_o = part[pl.ds(orow(my), SQ), :].astype(jnp.float32)
                new_rsr.wait_recv()
                rsb_r[t] = (rsb_r[t].astype(jnp.float32)
                            + add_e).astype(jnp.bfloat16)
                new_rsl.wait_recv()
                rsb_l[t] = (rsb_l[t].astype(jnp.float32)
                            + add_o).astype(jnp.bfloat16)

            if h < N_DEV - 1:
                new_agr.wait_send()
                new_agl.wait_send()
                agr_d, agl_d = new_agr, new_agl
            if h >= 2:
                new_rsr.wait_send()
                new_rsl.wait_send()

        out_ref[0, :, :] = rsb_r[N_DEV - 2].astype(jnp.float32)
        out_ref[1, :, :] = rsb_l[N_DEV - 2].astype(jnp.float32)

    return pl.pallas_call(
        body,
        out_shape=jax.ShapeDtypeStruct((B_LOC, SQ, D), jnp.float32),
        in_specs=[pl.BlockSpec(memory_space=pltpu.VMEM)] * 5,
        out_specs=pl.BlockSpec(memory_space=pltpu.VMEM),
        scratch_shapes=[
            pltpu.VMEM((ROWS, D), jnp.bfloat16),
            pltpu.VMEM((ROWS, D), jnp.bfloat16),
            pltpu.VMEM((N_DEV - 1, SQ, D), jnp.bfloat16),
            pltpu.VMEM((N_DEV - 1, SQ, D), jnp.bfloat16),
            pltpu.SemaphoreType.DMA((N_DEV - 1,)),
            pltpu.SemaphoreType.DMA((N_DEV - 1,)),
            pltpu.SemaphoreType.DMA((N_DEV - 1,)),
            pltpu.SemaphoreType.DMA((N_DEV - 1,)),
            pltpu.SemaphoreType.DMA((N_DEV - 1,)),
            pltpu.SemaphoreType.DMA((N_DEV - 1,)),
            pltpu.SemaphoreType.DMA((N_DEV - 1,)),
            pltpu.SemaphoreType.DMA((N_DEV - 1,)),
        ],
        compiler_params=pltpu.CompilerParams(
            collective_id=0,
            vmem_limit_bytes=63 * 1024 * 1024,
        ),
    )(x, Wq, Wk, Wv, Wo)
